# Initial kernel scaffold; baseline (speedup 1.0000x reference)
import jax
import jax.numpy as jnp
from jax.experimental import pallas as pl


def kernel(key, val, device, modality_table):
    raise NotImplementedError("write your pallas kernel here")


# speedup vs baseline: 1.4164x; 1.4164x over previous
import jax
import jax.numpy as jnp
from jax.experimental import pallas as pl

_MODALITY_IDX = 2
_BLOCK_ROWS = 512


def _add_row_kernel(val_ref, table_ref, out_ref):
    row = table_ref[_MODALITY_IDX, :]
    out_ref[...] = val_ref[...] + row[None, :]


def kernel(key, val, device, modality_table):
    b, s, d = val.shape
    flat = val.reshape(b * s, d)
    n = b * s
    grid = (n // _BLOCK_ROWS,)
    out = pl.pallas_call(
        _add_row_kernel,
        grid=grid,
        in_specs=[
            pl.BlockSpec((_BLOCK_ROWS, d), lambda i: (i, 0)),
            pl.BlockSpec(modality_table.shape, lambda i: (0, 0)),
        ],
        out_specs=pl.BlockSpec((_BLOCK_ROWS, d), lambda i: (i, 0)),
        out_shape=jax.ShapeDtypeStruct((n, d), val.dtype),
    )(flat, modality_table)
    return out.reshape(b, s, d)


# speedup vs baseline: 1.5440x; 1.0901x over previous
import jax
import jax.numpy as jnp
from jax.experimental import pallas as pl

_MODALITY_IDX = 2
_BLOCK_ROWS = 1024


def _add_row_kernel(val_ref, table_ref, out_ref):
    row = table_ref[_MODALITY_IDX, :]
    out_ref[...] = val_ref[...] + row[None, :]


def kernel(key, val, device, modality_table):
    b, s, d = val.shape
    flat = val.reshape(b * s, d)
    n = b * s
    grid = (n // _BLOCK_ROWS,)
    out = pl.pallas_call(
        _add_row_kernel,
        grid=grid,
        in_specs=[
            pl.BlockSpec((_BLOCK_ROWS, d), lambda i: (i, 0)),
            pl.BlockSpec(modality_table.shape, lambda i: (0, 0)),
        ],
        out_specs=pl.BlockSpec((_BLOCK_ROWS, d), lambda i: (i, 0)),
        out_shape=jax.ShapeDtypeStruct((n, d), val.dtype),
    )(flat, modality_table)
    return out.reshape(b, s, d)


# speedup vs baseline: 1.6016x; 1.0373x over previous
import jax
import jax.numpy as jnp
from jax.experimental import pallas as pl

_MODALITY_IDX = 2
_BLOCK_ROWS = 2048


def _add_row_kernel(val_ref, table_ref, out_ref):
    row = table_ref[_MODALITY_IDX, :]
    out_ref[...] = val_ref[...] + row[None, :]


def kernel(key, val, device, modality_table):
    b, s, d = val.shape
    flat = val.reshape(b * s, d)
    n = b * s
    grid = (n // _BLOCK_ROWS,)
    out = pl.pallas_call(
        _add_row_kernel,
        grid=grid,
        in_specs=[
            pl.BlockSpec((_BLOCK_ROWS, d), lambda i: (i, 0)),
            pl.BlockSpec(modality_table.shape, lambda i: (0, 0)),
        ],
        out_specs=pl.BlockSpec((_BLOCK_ROWS, d), lambda i: (i, 0)),
        out_shape=jax.ShapeDtypeStruct((n, d), val.dtype),
    )(flat, modality_table)
    return out.reshape(b, s, d)
